# baseline (speedup 1.0000x reference)
import functools

import jax
import jax.numpy as jnp
from jax import lax
from jax.experimental import pallas as pl
from jax.experimental.pallas import tpu as pltpu
from jax.experimental.pallas import tpu_sc as plsc

B, C, H, W = 16, 256, 32, 32
HW = H * W
N = B * HW
K = 1024
D = C

S = 4
SB = B // S
SN = SB * HW

NC = 2
NS = 16
NW = NC * NS
ROWS_PER_W = SN // NW


def _tc_body(emb_ref, z_ref, cin_ref, ain_ref,
             idx_ref, cout_ref, aout_ref, stats_ref):
    b = pl.program_id(0)
    emb = emb_ref[...]
    zb = z_ref[0]
    s_t = jax.lax.dot_general(
        emb, zb, (((1,), (0,)), ((), ())),
        preferred_element_type=jnp.float32,
        precision=lax.Precision.DEFAULT,
    )
    enorm = jnp.sum(emb * emb, axis=1, keepdims=True)
    znorm = jnp.sum(zb * zb, axis=0, keepdims=True)
    dist_t = (znorm - 2.0 * s_t) + enorm
    m = jnp.min(dist_t, axis=0, keepdims=True)
    ks = lax.broadcasted_iota(jnp.int32, (K, HW), 0)
    eq = dist_t == m
    idx = jnp.min(jnp.where(eq, ks, K), axis=0).astype(jnp.int32)
    idx_ref[0, 0, :] = idx

    @pl.when(b == 0)
    def _init():
        cout_ref[...] = cin_ref[...]
        aout_ref[0] = ain_ref[0]

    ones = jnp.ones((HW, 1), jnp.float32)
    cout_ref[...] += jax.lax.dot_general(
        eq.astype(jnp.float32), ones, (((1,), (0,)), ((), ())),
        preferred_element_type=jnp.float32)
    esum = jnp.sum(emb, axis=0, keepdims=True)
    zsum = jnp.sum(zb, axis=1, keepdims=True)
    cross = jax.lax.dot_general(
        esum, zsum, (((1,), (0,)), ((), ())),
        preferred_element_type=jnp.float32,
        precision=lax.Precision.HIGHEST)
    aout_ref[0] += (K * jnp.sum(znorm) + HW * jnp.sum(enorm)
                    - 2.0 * cross[0, 0])

    @pl.when(b == SB - 1)
    def _finalize():
        e_mean = cout_ref[...] * (1.0 / N)
        ent = jnp.sum(e_mean * jnp.log(e_mean + 1e-10))
        stats_ref[0] = jnp.exp(-ent)
        stats_ref[1] = aout_ref[0] * (1.0 / (N * K))


_tc_call = pl.pallas_call(
    _tc_body,
    grid=(SB,),
    in_specs=[
        pl.BlockSpec((K, D), lambda b: (0, 0)),
        pl.BlockSpec((1, C, HW), lambda b: (b, 0, 0)),
        pl.BlockSpec((K, 1), lambda b: (0, 0)),
        pl.BlockSpec(memory_space=pltpu.SMEM),
    ],
    out_specs=[
        pl.BlockSpec((1, 1, HW), lambda b: (b, 0, 0)),
        pl.BlockSpec((K, 1), lambda b: (0, 0)),
        pl.BlockSpec(memory_space=pltpu.SMEM),
        pl.BlockSpec(memory_space=pltpu.SMEM),
    ],
    out_shape=[
        jax.ShapeDtypeStruct((SB, 1, HW), jnp.int32),
        jax.ShapeDtypeStruct((K, 1), jnp.float32),
        jax.ShapeDtypeStruct((1,), jnp.float32),
        jax.ShapeDtypeStruct((2,), jnp.float32),
    ],
)


def _sc_gather_body(emb_hbm, idx_hbm, out_hbm, idx_v, buf, sem):
    c = lax.axis_index("c")
    s = lax.axis_index("s")
    wid = s * NC + c
    base = wid * ROWS_PER_W
    pltpu.sync_copy(idx_hbm.at[pl.ds(base, ROWS_PER_W)], idx_v)
    pltpu.async_copy(emb_hbm.at[idx_v], buf, sem).wait()
    pltpu.sync_copy(buf, out_hbm.at[pl.ds(base, ROWS_PER_W)])


@functools.lru_cache(maxsize=1)
def _make_sc_gather():
    return pl.kernel(
        _sc_gather_body,
        out_type=jax.ShapeDtypeStruct((SN, D), jnp.float32),
        mesh=plsc.VectorSubcoreMesh(
            core_axis_name="c", subcore_axis_name="s",
            num_cores=NC, num_subcores=NS),
        scratch_types=[
            pltpu.VMEM((ROWS_PER_W,), jnp.int32),
            pltpu.VMEM((ROWS_PER_W, D), jnp.float32),
            pltpu.SemaphoreType.DMA,
        ],
    )


def kernel(z, embedding):
    zs = z.reshape(B, C, HW)
    counts = jnp.zeros((K, 1), jnp.float32)
    acc = jnp.zeros((1,), jnp.float32)
    stats = None
    idx_slices = []
    zq_slices = []
    sc_gather = _make_sc_gather()
    for s in range(S):
        z_sl = lax.slice_in_dim(zs, s * SB, (s + 1) * SB, axis=0)
        idx3, counts, acc, stats = _tc_call(embedding, z_sl, counts, acc)
        idx_slices.append(idx3)
        zq_slices.append(sc_gather(embedding, idx3.reshape(SN)))
    zq_flat = jnp.concatenate(zq_slices, axis=0)
    z_q = zq_flat.reshape(B, HW, C).transpose(0, 2, 1).reshape(B, C, H, W)
    loss = jnp.zeros((), jnp.float32)
    indices = jnp.concatenate(idx_slices, axis=0).reshape(B, H, W)
    return (z_q, loss, stats[0], indices, stats[1])
